# baseline (device time: 423407 ns/iter reference)
import jax
import jax.numpy as jnp
from jax import lax
from jax.experimental import pallas as pl
from jax.experimental.pallas import tpu as pltpu

NC = 32
NLOC = 8


def kernel(x):
    _, m, n = x.shape
    r = m // NC

    def body(
        x_ref, out_ref, xloc, rxbuf, red,
        xsend, xrecv, ysend, yrecv, ld_sem, st_sem,
    ):
        my_x = lax.axis_index("x")
        my_y = lax.axis_index("y")
        x_nbr = (1 - my_x, my_y)
        y_nbr = (my_x, 1 - my_y)
        col0 = my_y * n

        barrier = pltpu.get_barrier_semaphore()
        for nbr in (x_nbr, y_nbr):
            pl.semaphore_signal(
                barrier, inc=1, device_id=nbr,
                device_id_type=pl.DeviceIdType.MESH,
            )
        pl.semaphore_wait(barrier, 2)

        def rows(c):
            return pl.ds(c * r, r)

        def start_load(c):
            d = pltpu.make_async_copy(
                x_ref.at[0, rows(c), :], xloc.at[c % NLOC], ld_sem.at[c % NLOC]
            )
            d.start()
            return d

        def start_send(c):
            d = pltpu.make_async_remote_copy(
                src_ref=xloc.at[c % NLOC],
                dst_ref=rxbuf.at[c],
                send_sem=xsend.at[c],
                recv_sem=xrecv.at[c],
                device_id=x_nbr,
                device_id_type=pl.DeviceIdType.MESH,
            )
            d.start()
            return d

        ld = {c: start_load(c) for c in range(min(NLOC, NC))}
        x_rdmas = {}
        for c in range(min(NLOC - 1, NC)):
            ld[c].wait()
            x_rdmas[c] = start_send(c)

        st = {}
        y_rdmas = {}
        for c in range(NC):
            s = c % 2
            if c + NLOC - 1 < NC:
                ld[c + NLOC - 1].wait()
                x_rdmas[c + NLOC - 1] = start_send(c + NLOC - 1)
            x_rdmas[c].wait_recv()
            if c >= 2:
                st[c - 2].wait()
                y_rdmas[c - 2].wait_send()
            red[s] = xloc[c % NLOC] + rxbuf[c]
            if c + NLOC < NC:
                x_rdmas[c].wait_send()
                ld[c + NLOC] = start_load(c + NLOC)
            st[c] = pltpu.make_async_copy(
                red.at[s], out_ref.at[rows(c), pl.ds(col0, n)], st_sem.at[s]
            )
            st[c].start()
            y_rdmas[c] = pltpu.make_async_remote_copy(
                src_ref=red.at[s],
                dst_ref=out_ref.at[rows(c), pl.ds(col0, n)],
                send_sem=ysend.at[c],
                recv_sem=yrecv.at[c],
                device_id=y_nbr,
                device_id_type=pl.DeviceIdType.MESH,
            )
            y_rdmas[c].start()

        for c in range(max(0, NC - 2), NC):
            st[c].wait()
            y_rdmas[c].wait_send()
        for c in range(max(0, NC - NLOC), NC):
            x_rdmas[c].wait_send()
        for c in range(NC):
            y_rdmas[c].wait_recv()

    return pl.pallas_call(
        body,
        out_shape=jax.ShapeDtypeStruct((m, 2 * n), x.dtype),
        in_specs=[pl.BlockSpec(memory_space=pl.ANY)],
        out_specs=pl.BlockSpec(memory_space=pl.ANY),
        scratch_shapes=[
            pltpu.VMEM((NLOC, r, n), x.dtype),
            pltpu.VMEM((NC, r, n), x.dtype),
            pltpu.VMEM((2, r, n), x.dtype),
            pltpu.SemaphoreType.DMA((NC,)),
            pltpu.SemaphoreType.DMA((NC,)),
            pltpu.SemaphoreType.DMA((NC,)),
            pltpu.SemaphoreType.DMA((NC,)),
            pltpu.SemaphoreType.DMA((NLOC,)),
            pltpu.SemaphoreType.DMA((2,)),
        ],
        compiler_params=pltpu.CompilerParams(
            collective_id=0, vmem_limit_bytes=64 * 1024 * 1024
        ),
    )(x)
